# Initial kernel scaffold; baseline (speedup 1.0000x reference)
import jax
import jax.numpy as jnp
from jax.experimental import pallas as pl


def kernel(x, table):
    raise NotImplementedError("write your pallas kernel here")


# speedup vs baseline: 1.1010x; 1.1010x over previous
import functools

import jax
import jax.numpy as jnp
from jax import lax
from jax.experimental import pallas as pl
from jax.experimental.pallas import tpu as pltpu
from jax.experimental.pallas import tpu_sc as plsc

POS_EMB_SIZE = 8193
D_WORD_VEC = 1024
BATCH = 4
MAX_LEN = 8192

_NUM_CORES = 2
_NUM_SUBCORES = 16
_NUM_WORKERS = _NUM_CORES * _NUM_SUBCORES
_ROWS_PER_WORKER = MAX_LEN // _NUM_WORKERS
_CHUNK = 32
_NUM_CHUNKS = _ROWS_PER_WORKER // _CHUNK


_CHUNK_WORDS = _CHUNK * D_WORD_VEC


@functools.partial(
    pl.kernel,
    mesh=plsc.VectorSubcoreMesh(core_axis_name="c", subcore_axis_name="s"),
    out_type=jax.ShapeDtypeStruct((BATCH * MAX_LEN * D_WORD_VEC,), jnp.float32),
    scratch_types=[
        pltpu.VMEM((_CHUNK_WORDS,), jnp.float32),
        pltpu.VMEM((_CHUNK_WORDS,), jnp.float32),
        pltpu.SemaphoreType.DMA,
        pltpu.SemaphoreType.DMA,
        pltpu.SemaphoreType.DMA,
        pltpu.SemaphoreType.DMA,
    ],
)
def _sc_broadcast_rows(table_hbm, out_hbm, buf0, buf1, rsem0, rsem1, wsem0, wsem1):
    wid = lax.axis_index("s") * _NUM_CORES + lax.axis_index("c")
    base = wid * _ROWS_PER_WORKER

    bufs = (buf0, buf1)
    rsems = (rsem0, rsem1)
    wsems = (wsem0, wsem1)
    pending_writes = [None, None]

    def read_copy(i):
        row0 = base + i * _CHUNK
        return pltpu.make_async_copy(
            table_hbm.at[pl.ds((row0 + 1) * D_WORD_VEC, _CHUNK_WORDS)],
            bufs[i % 2],
            rsems[i % 2],
        )

    read_copy(0).start()

    for i in range(_NUM_CHUNKS):
        slot = i % 2
        row0 = base + i * _CHUNK
        read_copy(i).wait()
        if pending_writes[1 - slot] is not None:
            for cp in pending_writes[1 - slot]:
                cp.wait()
            pending_writes[1 - slot] = None
        if i + 1 < _NUM_CHUNKS:
            read_copy(i + 1).start()
        writes = []
        for b in range(BATCH):
            cp = pltpu.make_async_copy(
                bufs[slot],
                out_hbm.at[pl.ds((b * MAX_LEN + row0) * D_WORD_VEC, _CHUNK_WORDS)],
                wsems[slot],
            )
            cp.start()
            writes.append(cp)
        pending_writes[slot] = writes

    for slot in range(2):
        if pending_writes[slot] is not None:
            for cp in pending_writes[slot]:
                cp.wait()


def kernel(x, table):
    del x
    flat = _sc_broadcast_rows(table.reshape(-1))
    return flat.reshape(BATCH, MAX_LEN, D_WORD_VEC)
